# baseline (speedup 1.0000x reference)
import functools
import numpy as np
import jax
import jax.numpy as jnp
from jax import lax
from jax.experimental import pallas as pl
from jax.experimental.pallas import tpu as pltpu
from jax.experimental.pallas import tpu_sc as plsc

N = 10000
E = 320000
D_IN = 128
D_OUT = 128
RATE_NODE = 0.5
NOISE_NODE = 0.05
RATE_EDGE = 0.5
NOISE_EDGE = 0.05
ALPHA = 0.5
SCE_ALPHA = 2.0
N_ADD_MAX = int(NOISE_EDGE * E)
E2 = E + N_ADD_MAX + N

NZPAD = 128
NP = N + NZPAD
BATCH = 128
ROWS_PER_TILE = N // 16
WIN = 16
NBC = WIN * (-(-E2 // (16 * BATCH * WIN)))
EPC = 16 * BATCH * NBC


def _const_tables():
    rng = np.random.RandomState(0)
    perm = rng.permutation(N)
    num_mask = int(RATE_NODE * N)
    mask_nodes = perm[:num_mask]
    perm_mask = rng.permutation(num_mask)
    n_noise = int(NOISE_NODE * num_mask)
    token_nodes = mask_nodes[perm_mask[:int((1 - NOISE_NODE) * num_mask)]]
    noise_nodes = mask_nodes[perm_mask[num_mask - n_noise:]]
    noise_chosen = rng.permutation(N)[:n_noise]
    srcmap = np.arange(N, dtype=np.int32)
    srcmap[noise_nodes] = noise_chosen.astype(np.int32)
    token_flag = np.zeros((N,), np.bool_)
    token_flag[token_nodes] = True

    rng1 = np.random.RandomState(1)
    u_keep = rng1.random_sample(E) >= RATE_EDGE
    rng2 = np.random.RandomState(1)
    words = rng2.randint(0, 2 ** 32, size=2 * E + 1000000, dtype=np.uint32)
    mask_bits = (1 << (N - 1).bit_length()) - 1
    vals = (words[1::2] & mask_bits).astype(np.int64)
    acc = vals <= (N - 1)
    accepted = vals[acc].astype(np.int32)
    cumacc = np.concatenate([[0], np.cumsum(acc)]).astype(np.int32)
    accepted = np.concatenate([accepted, np.zeros(2 * N_ADD_MAX, np.int32)])
    return token_flag, srcmap, u_keep, accepted, cumacc


_TOKEN_FLAG, _SRCMAP, _U_KEEP, _ACCEPTED, _CUMACC = _const_tables()


def _mask_edges_fast(src, dst):
    valid = src != dst
    cs = jnp.cumsum(valid.astype(jnp.int32))
    rank = jnp.clip(cs - 1, 0, E - 1)
    keep2 = valid & jnp.asarray(_U_KEEP)[rank]
    w1 = keep2.astype(jnp.float32)
    k = cs[-1]
    m = keep2.sum().astype(jnp.int32)
    n_add = (m * int(NOISE_EDGE * 100)) // 100
    start = jnp.asarray(_CUMACC)[k]
    idx = jnp.arange(N_ADD_MAX, dtype=jnp.int32)
    act = idx < n_add
    acc_t = jnp.asarray(_ACCEPTED)
    add_s = jnp.where(act, acc_t[start + idx], 0)
    add_d = jnp.where(act, acc_t[start + n_add + idx], 0)
    w2 = act.astype(jnp.float32)
    loops = jnp.arange(N, dtype=jnp.int32)
    s = jnp.concatenate([src.astype(jnp.int32), add_s, loops])
    d = jnp.concatenate([dst.astype(jnp.int32), add_d, loops])
    w = jnp.concatenate([w1, w2, jnp.ones((N,), jnp.float32)])
    return s, d, w


def _prelu(x, a):
    return jnp.where(x >= 0, x, a * x)


mesh = plsc.VectorSubcoreMesh(core_axis_name="c", subcore_axis_name="s")


@functools.partial(
    pl.kernel,
    mesh=mesh,
    out_type=jax.ShapeDtypeStruct((32, ROWS_PER_TILE, D_IN), jnp.float32),
    scratch_types=[
        pltpu.VMEM((WIN, BATCH), jnp.int32),
        pltpu.VMEM((WIN, BATCH), jnp.int32),
        pltpu.VMEM((BATCH, D_IN), jnp.float32),
        pltpu.VMEM((BATCH, D_IN), jnp.float32),
        pltpu.VMEM_SHARED((N, D_IN), jnp.float32),
        pltpu.SemaphoreType.DMA,
        pltpu.SemaphoreType.DMA,
    ],
)
def _agg_kernel(tables_hbm, srcx_hbm, dstx_hbm, zeros_hbm, out_hbm,
                sidx, didx, rows0, rows1, acc, sem0, sem1):
    c = lax.axis_index("c")
    s = lax.axis_index("s")
    wid = c * 16 + s
    tbl = tables_hbm.at[c]
    rows = (rows0, rows1)
    sems = (sem0, sem1)

    pltpu.sync_copy(zeros_hbm, acc.at[pl.ds(s * ROWS_PER_TILE, ROWS_PER_TILE)])
    plsc.subcore_barrier()

    for w in range(NBC // WIN):
        pltpu.sync_copy(srcx_hbm.at[wid].at[pl.ds(w * WIN, WIN)], sidx)
        pltpu.sync_copy(dstx_hbm.at[wid].at[pl.ds(w * WIN, WIN)], didx)
        h = pltpu.async_copy(tbl.at[sidx.at[0]], rows[0], sems[0])
        for b in range(WIN):
            p = b % 2
            if b + 1 < WIN:
                hn = pltpu.async_copy(tbl.at[sidx.at[b + 1]],
                                      rows[(b + 1) % 2], sems[(b + 1) % 2])
            h.wait()
            pltpu.sync_copy(rows[p], acc.at[didx.at[b]], add=True)
            if b + 1 < WIN:
                h = hn

    plsc.subcore_barrier()
    pltpu.sync_copy(
        acc.at[pl.ds(s * ROWS_PER_TILE, ROWS_PER_TILE)],
        out_hbm.at[wid])


_AGG_ZEROS = np.zeros((ROWS_PER_TILE, D_IN), np.float32)


def _pad_edges(sx, dx):
    n = sx.shape[0]
    pad = EPC - n
    fill = N + (jnp.arange(pad, dtype=jnp.int32) & (NZPAD - 1))
    shape = (16, NBC, BATCH)
    sxp = jnp.concatenate([sx, fill]).reshape(shape)
    dxp = jnp.concatenate([dx, jnp.zeros((pad,), jnp.int32)]).reshape(shape)
    return sxp, dxp


def _agg(tables, sxp, dxp):
    out = _agg_kernel(tables, sxp, dxp, jnp.asarray(_AGG_ZEROS))
    return out.reshape(2, N, D_IN)


def _enc_body(tail, part_ref, sin_ref, sout_ref, w_ref, vec_ref, out_ref):
    for g in range(2):
        agg = part_ref[g] * sin_ref[g]
        ac = vec_ref[3, 0]
        aa = vec_ref[4, 0]
        h = lax.dot_general(agg, w_ref[...], (((1,), (0,)), ((), ())),
                            preferred_element_type=jnp.float32)
        h = _prelu(h + vec_ref[0, :][None, :], ac)
        mu = jnp.mean(h, axis=0, keepdims=True)
        hc = h - mu
        var = jnp.mean(hc * hc, axis=0, keepdims=True)
        h = hc / jnp.sqrt(var + 1e-5) * vec_ref[1, :][None, :] \
            + vec_ref[2, :][None, :]
        h = _prelu(h, aa)
        if tail == 'scale':
            h = h * sout_ref[g]
        out_ref[g, :N, :] = h
        out_ref[g, N:, :] = jnp.zeros((NZPAD, D_IN), jnp.float32)


def _enc_dense(tail, part, sin, sout, L):
    vec = jnp.stack([
        L['b'], L['g'], L['be'],
        jnp.full((D_OUT,), L['ac'], jnp.float32),
        jnp.full((D_OUT,), L['aa'], jnp.float32),
    ])
    return pl.pallas_call(
        functools.partial(_enc_body, tail),
        out_shape=jax.ShapeDtypeStruct((2, NP, D_IN), jnp.float32),
    )(part, sin, sout, L['W'], vec)


def _dec_body(part_ref, w1_ref, vec1_ref, w2_ref, vec2_ref, x_ref,
              l1_ref, h2_ref):
    outs = []
    for g, (w_ref, vec_ref) in enumerate(((w1_ref, vec1_ref),
                                          (w2_ref, vec2_ref))):
        h = lax.dot_general(part_ref[g], w_ref[...], (((1,), (0,)), ((), ())),
                            preferred_element_type=jnp.float32)
        h = _prelu(h + vec_ref[0, :][None, :], vec_ref[3, 0])
        mu = jnp.mean(h, axis=0, keepdims=True)
        hc = h - mu
        var = jnp.mean(hc * hc, axis=0, keepdims=True)
        h = hc / jnp.sqrt(var + 1e-5) * vec_ref[1, :][None, :] \
            + vec_ref[2, :][None, :]
        outs.append(_prelu(h, vec_ref[4, 0]))
    h = outs[0]
    hn = h / jnp.clip(jnp.sqrt(jnp.sum(h * h, axis=1, keepdims=True)),
                      1e-12, None)
    x = x_ref[...]
    xn = x / jnp.clip(jnp.sqrt(jnp.sum(x * x, axis=1, keepdims=True)),
                      1e-12, None)
    cos = jnp.sum(hn * xn, axis=1)
    l1_ref[...] = jnp.mean((1.0 - cos) ** SCE_ALPHA).reshape(1, 1)
    h = outs[1]
    h2_ref[...] = h / jnp.clip(jnp.sqrt(jnp.sum(h * h, axis=1, keepdims=True)),
                               1e-12, None)


def _dec_dense(part, L1, L2, x):
    def vecs(L):
        return jnp.stack([
            L['b'], L['g'], L['be'],
            jnp.full((D_OUT,), L['ac'], jnp.float32),
            jnp.full((D_OUT,), L['aa'], jnp.float32),
        ])
    return pl.pallas_call(
        _dec_body,
        out_shape=[jax.ShapeDtypeStruct((1, 1), jnp.float32),
                   jax.ShapeDtypeStruct((N, D_IN), jnp.float32)],
    )(part, L1['W'], vecs(L1), L2['W'], vecs(L2), x)


def _prep_body(xm_ref, x_ref, s1_ref, s2_ref, t_ref):
    t_ref[0, :N, :] = xm_ref[...] * s1_ref[...]
    t_ref[0, N:, :] = jnp.zeros((NZPAD, D_IN), jnp.float32)
    t_ref[1, :N, :] = x_ref[...] * s2_ref[...]
    t_ref[1, N:, :] = jnp.zeros((NZPAD, D_IN), jnp.float32)


def _prep(xm, x, s1, s2):
    return pl.pallas_call(
        _prep_body,
        out_shape=jax.ShapeDtypeStruct((2, NP, D_IN), jnp.float32),
    )(xm, x, s1, s2)


_BI = 400


def _loss2_body(hi_ref, h_ref, a_ref, out_ref):
    p = lax.dot_general(hi_ref[...], h_ref[...], (((1,), (1,)), ((), ())),
                        preferred_element_type=jnp.float32)
    d = p - a_ref[...]
    part = jnp.sum(d * d).reshape(1, 1)

    @pl.when(pl.program_id(0) == 0)
    def _():
        out_ref[...] = jnp.zeros((1, 1), jnp.float32)

    out_ref[...] += part


def _loss2(h2, matrix):
    gi = N // _BI
    total = pl.pallas_call(
        _loss2_body,
        grid=(gi,),
        in_specs=[
            pl.BlockSpec((_BI, D_IN), lambda i: (i, 0)),
            pl.BlockSpec((N, D_IN), lambda i: (0, 0)),
            pl.BlockSpec((_BI, N), lambda i: (i, 0)),
        ],
        out_specs=pl.BlockSpec((1, 1), lambda i: (0, 0)),
        out_shape=jax.ShapeDtypeStruct((1, 1), jnp.float32),
    )(h2, h2, matrix)
    return total[0, 0] / (float(N) * float(N))


def kernel(x, edge_index, matrix, params):
    src, dst = edge_index[0].astype(jnp.int32), edge_index[1].astype(jnp.int32)
    msrc, mdst, mw = _mask_edges_fast(src, dst)

    token = jnp.asarray(_TOKEN_FLAG)[:, None]
    xm = jnp.where(token, params['mask_token'][0][None, :], x[jnp.asarray(_SRCMAP)])

    deg_o1 = jnp.clip(jnp.zeros((N,), jnp.float32).at[src].add(1.0), 1.0, None)
    deg_i1 = jnp.clip(jnp.zeros((N,), jnp.float32).at[dst].add(1.0), 1.0, None)
    deg_o2 = jnp.clip(jnp.zeros((N,), jnp.float32).at[msrc].add(mw), 1.0, None)
    deg_i2 = jnp.clip(jnp.zeros((N,), jnp.float32).at[mdst].add(mw), 1.0, None)
    so1 = (deg_o1 ** -0.5)[:, None]
    so2 = (deg_o2 ** -0.5)[:, None]
    sin = jnp.stack([(deg_i1 ** -0.5)[:, None], (deg_i2 ** -0.5)[:, None]])
    sout = jnp.stack([so1, so2])
    ones = jnp.ones((2, N, 1), jnp.float32)

    sx1, dx1 = _pad_edges(src, dst)
    active = mw > 0.0
    msrc_r = jnp.where(active, msrc,
                       N + (jnp.arange(E2, dtype=jnp.int32) & (NZPAD - 1)))
    sx2, dx2 = _pad_edges(msrc_r, mdst)
    sxp = jnp.concatenate([sx1, sx2]).reshape(32, NBC, BATCH)
    dxp = jnp.concatenate([dx1, dx2]).reshape(32, NBC, BATCH)

    enc1, enc2 = params['enc']

    t = _prep(xm, x, so1, so2)
    p = _agg(t, sxp, dxp)
    t = _enc_dense('scale', p, sin, sout, enc1)
    p = _agg(t, sxp, dxp)
    t = _enc_dense('pad', p, sin, ones, enc2)
    p = _agg(t, sxp, dxp)
    loss1, h2n = _dec_dense(p, params['dec1'][0], params['dec2'][0], x)

    loss2 = _loss2(h2n, matrix)
    return ALPHA * loss1[0, 0] + (1.0 - ALPHA) * loss2
